# Initial kernel scaffold; baseline (speedup 1.0000x reference)
import jax
import jax.numpy as jnp
from jax.experimental import pallas as pl


def kernel(features, edge_index, W1, b1, W2, b2, Wc, bc):
    raise NotImplementedError("write your pallas kernel here")


# speedup vs baseline: 4.4477x; 4.4477x over previous
import functools

import jax
import jax.numpy as jnp
from jax import lax
from jax.experimental import pallas as pl
from jax.experimental.pallas import tpu as pltpu
from jax.experimental.pallas import tpu_sc as plsc

N = 10000
E = 320000
D = 128
NCLS = 16

NC = 2
NS = 16
NW = NC * NS
E_PER_W = E // NW
CHUNK = 80
N_CHUNKS = E_PER_W // CHUNK
NP = 10240
ROWS_PER_TILE = NP // NS
DEGW = 8


def _degrees_body(src_hbm, dst_hbm, z8_hbm, eo_hbm, ei_hbm, out_hbm,
                  sidx, didx, eo_v, ei_v, deg_sh):
    c = lax.axis_index("c")
    s = lax.axis_index("s")
    row0 = s * ROWS_PER_TILE
    pltpu.sync_copy(z8_hbm.at[pl.ds(row0, ROWS_PER_TILE)],
                    deg_sh.at[pl.ds(row0, ROWS_PER_TILE)])
    pltpu.sync_copy(eo_hbm, eo_v)
    pltpu.sync_copy(ei_hbm, ei_v)
    plsc.subcore_barrier()

    base = (c * NS + s) * E_PER_W

    def body(j, carry):
        off = base + j * CHUNK
        pltpu.sync_copy(src_hbm.at[pl.ds(off, CHUNK)], sidx)
        pltpu.sync_copy(dst_hbm.at[pl.ds(off, CHUNK)], didx)
        pltpu.sync_copy(eo_v, deg_sh.at[sidx], add=True)
        pltpu.sync_copy(ei_v, deg_sh.at[didx], add=True)
        return carry

    lax.fori_loop(0, N_CHUNKS, body, 0)
    plsc.subcore_barrier()
    pltpu.sync_copy(deg_sh.at[pl.ds(row0, ROWS_PER_TILE)],
                    out_hbm.at[c, pl.ds(row0, ROWS_PER_TILE)])


def _segment_sum_body(h_hbm, src_hbm, dst_hbm, z_hbm, out_hbm,
                      sidx, didx, rows, agg_sh, sem):
    c = lax.axis_index("c")
    s = lax.axis_index("s")
    row0 = s * ROWS_PER_TILE
    pltpu.sync_copy(z_hbm.at[pl.ds(row0, ROWS_PER_TILE)],
                    agg_sh.at[pl.ds(row0, ROWS_PER_TILE)])
    plsc.subcore_barrier()

    base = (c * NS + s) * E_PER_W

    def body(j, carry):
        off = base + j * CHUNK
        pltpu.sync_copy(src_hbm.at[pl.ds(off, CHUNK)], sidx)
        pltpu.sync_copy(dst_hbm.at[pl.ds(off, CHUNK)], didx)
        pltpu.async_copy(h_hbm.at[sidx], rows, sem).wait()
        pltpu.sync_copy(rows, agg_sh.at[didx], add=True)
        return carry

    lax.fori_loop(0, N_CHUNKS, body, 0)
    plsc.subcore_barrier()
    pltpu.sync_copy(agg_sh.at[pl.ds(row0, ROWS_PER_TILE)],
                    out_hbm.at[c, pl.ds(row0, ROWS_PER_TILE)])


@functools.cache
def _sc_kernels():
    mesh = plsc.VectorSubcoreMesh(core_axis_name="c", subcore_axis_name="s")
    degrees = functools.partial(
        pl.kernel,
        mesh=mesh,
        out_type=jax.ShapeDtypeStruct((NC, NP, DEGW), jnp.float32),
        scratch_types=[
            pltpu.VMEM((CHUNK,), jnp.int32),
            pltpu.VMEM((CHUNK,), jnp.int32),
            pltpu.VMEM((CHUNK, DEGW), jnp.float32),
            pltpu.VMEM((CHUNK, DEGW), jnp.float32),
            pltpu.VMEM_SHARED((NP, DEGW), jnp.float32),
        ],
    )(_degrees_body)
    segment_sum = functools.partial(
        pl.kernel,
        mesh=mesh,
        out_type=jax.ShapeDtypeStruct((NC, NP, D), jnp.float32),
        scratch_types=[
            pltpu.VMEM((CHUNK,), jnp.int32),
            pltpu.VMEM((CHUNK,), jnp.int32),
            pltpu.VMEM((CHUNK, D), jnp.float32),
            pltpu.VMEM_SHARED((NP, D), jnp.float32),
            pltpu.SemaphoreType.DMA,
        ],
    )(_segment_sum_body)
    return degrees, segment_sum


_R = 2000


def _mm1_body(x_ref, w_ref, degp_ref, o_ref):
    d = degp_ref[0] + degp_ref[1]
    so = lax.rsqrt(jnp.maximum(d[:, 0:1], 1.0))
    o_ref[...] = jnp.dot(x_ref[...] * so, w_ref[...],
                         preferred_element_type=jnp.float32)


def _mm2_body(aggp_ref, degp_ref, b_ref, w_ref, o_ref):
    agg = aggp_ref[0] + aggp_ref[1]
    d = degp_ref[0] + degp_ref[1]
    si = lax.rsqrt(jnp.maximum(d[:, 1:2], 1.0))
    so = lax.rsqrt(jnp.maximum(d[:, 0:1], 1.0))
    h = jnp.maximum(agg * si + b_ref[...], 0.0)
    o_ref[...] = jnp.dot(h * so, w_ref[...],
                         preferred_element_type=jnp.float32)


def _final_body(aggp_ref, degp_ref, b_ref, wc_ref, bc_ref,
                logits_ref, hg_ref, acc_ref):
    i = pl.program_id(0)
    agg = aggp_ref[0] + aggp_ref[1]
    d = degp_ref[0] + degp_ref[1]
    si = lax.rsqrt(jnp.maximum(d[:, 1:2], 1.0))
    h = jnp.maximum(agg * si + b_ref[...], 0.0)
    part = jnp.sum(h, axis=0, keepdims=True)

    @pl.when(i == 0)
    def _():
        acc_ref[...] = part

    @pl.when(i != 0)
    def _():
        acc_ref[...] = acc_ref[...] + part

    @pl.when(i == (N // _R) - 1)
    def _():
        hg = acc_ref[...] * (1.0 / N)
        hg_ref[...] = hg
        logits_ref[...] = jnp.dot(hg, wc_ref[...],
                                  preferred_element_type=jnp.float32) + bc_ref[...]


def kernel(features, edge_index, W1, b1, W2, b2, Wc, bc):
    _degrees, _segment_sum = _sc_kernels()
    src = edge_index[0]
    dst = edge_index[1]
    zD = jnp.zeros((NP, D), jnp.float32)
    z8 = jnp.zeros((NP, DEGW), jnp.float32)
    e_out = jnp.broadcast_to(
        jnp.array([1.0] + [0.0] * (DEGW - 1), jnp.float32), (CHUNK, DEGW))
    e_in = jnp.broadcast_to(
        jnp.array([0.0, 1.0] + [0.0] * (DEGW - 2), jnp.float32), (CHUNK, DEGW))

    degp = _degrees(src, dst, z8, e_out, e_in)

    grid = (N // _R,)
    mm1 = pl.pallas_call(
        _mm1_body,
        grid=grid,
        in_specs=[
            pl.BlockSpec((_R, D), lambda i: (i, 0)),
            pl.BlockSpec((D, D), lambda i: (0, 0)),
            pl.BlockSpec((NC, _R, DEGW), lambda i: (0, i, 0)),
        ],
        out_specs=pl.BlockSpec((_R, D), lambda i: (i, 0)),
        out_shape=jax.ShapeDtypeStruct((N, D), jnp.float32),
    )
    h1pre = mm1(features, W1, degp)

    aggp1 = _segment_sum(h1pre, src, dst, zD)

    mm2 = pl.pallas_call(
        _mm2_body,
        grid=grid,
        in_specs=[
            pl.BlockSpec((NC, _R, D), lambda i: (0, i, 0)),
            pl.BlockSpec((NC, _R, DEGW), lambda i: (0, i, 0)),
            pl.BlockSpec((1, D), lambda i: (0, 0)),
            pl.BlockSpec((D, D), lambda i: (0, 0)),
        ],
        out_specs=pl.BlockSpec((_R, D), lambda i: (i, 0)),
        out_shape=jax.ShapeDtypeStruct((N, D), jnp.float32),
    )
    h2pre = mm2(aggp1, degp, b1.reshape(1, D), W2)

    aggp2 = _segment_sum(h2pre, src, dst, zD)

    fin = pl.pallas_call(
        _final_body,
        grid=grid,
        in_specs=[
            pl.BlockSpec((NC, _R, D), lambda i: (0, i, 0)),
            pl.BlockSpec((NC, _R, DEGW), lambda i: (0, i, 0)),
            pl.BlockSpec((1, D), lambda i: (0, 0)),
            pl.BlockSpec((D, NCLS), lambda i: (0, 0)),
            pl.BlockSpec((1, NCLS), lambda i: (0, 0)),
        ],
        out_specs=[
            pl.BlockSpec((1, NCLS), lambda i: (0, 0)),
            pl.BlockSpec((1, D), lambda i: (0, 0)),
        ],
        out_shape=[
            jax.ShapeDtypeStruct((1, NCLS), jnp.float32),
            jax.ShapeDtypeStruct((1, D), jnp.float32),
        ],
        scratch_shapes=[pltpu.VMEM((1, D), jnp.float32)],
    )
    logits, hg = fin(aggp2, degp, b2.reshape(1, D), Wc, bc.reshape(1, NCLS))
    return (logits, hg)


# speedup vs baseline: 5.7811x; 1.2998x over previous
import functools

import jax
import jax.numpy as jnp
from jax import lax
from jax.experimental import pallas as pl
from jax.experimental.pallas import tpu as pltpu
from jax.experimental.pallas import tpu_sc as plsc

N = 10000
E = 320000
D = 128
NCLS = 16

NC = 2
NS = 16
NW = NC * NS
E_PER_W = E // NW
CHUNK = 80
N_CHUNKS = E_PER_W // CHUNK
NP = 10240
ROWS_PER_TILE = NP // NS
DEGW = 8


NBUF = 5
N_STEPS = N_CHUNKS // NBUF


def _degrees_body(src_hbm, dst_hbm, z8_hbm, eo_hbm, ei_hbm, out_hbm,
                  sidx, didx, eo_v, ei_v, deg_sh):
    c = lax.axis_index("c")
    s = lax.axis_index("s")
    w = c * NS + s
    row0 = s * ROWS_PER_TILE
    pltpu.sync_copy(z8_hbm.at[pl.ds(row0, ROWS_PER_TILE)],
                    deg_sh.at[pl.ds(row0, ROWS_PER_TILE)])
    pltpu.sync_copy(eo_hbm, eo_v)
    pltpu.sync_copy(ei_hbm, ei_v)
    plsc.subcore_barrier()

    def body(j, carry):
        pltpu.sync_copy(src_hbm.at[w, j], sidx)
        pltpu.sync_copy(dst_hbm.at[w, j], didx)
        pltpu.sync_copy(eo_v, deg_sh.at[sidx], add=True)
        pltpu.sync_copy(ei_v, deg_sh.at[didx], add=True)
        return carry

    lax.fori_loop(0, N_CHUNKS, body, 0)
    plsc.subcore_barrier()
    pltpu.sync_copy(deg_sh.at[pl.ds(row0, ROWS_PER_TILE)],
                    out_hbm.at[c, pl.ds(row0, ROWS_PER_TILE)])


def _segment_sum_body(h_hbm, src_hbm, dst_hbm, z_hbm, out_hbm,
                      sidx, didx, rows, agg_sh, sem_i):
    c = lax.axis_index("c")
    s = lax.axis_index("s")
    w = c * NS + s
    row0 = s * ROWS_PER_TILE
    pltpu.sync_copy(z_hbm.at[pl.ds(row0, ROWS_PER_TILE)],
                    agg_sh.at[pl.ds(row0, ROWS_PER_TILE)])
    pltpu.async_copy(src_hbm.at[w], sidx, sem_i)
    pltpu.async_copy(dst_hbm.at[w], didx, sem_i)
    pltpu.make_async_copy(src_hbm.at[w], sidx, sem_i).wait()
    pltpu.make_async_copy(dst_hbm.at[w], didx, sem_i).wait()
    plsc.subcore_barrier()

    def body(step, carry):
        j0 = step * NBUF
        for b in range(NBUF):
            pltpu.async_copy(h_hbm.at[sidx.at[j0 + b]], rows, sem_i).wait()
            pltpu.sync_copy(rows, agg_sh.at[didx.at[j0 + b]], add=True)
        return carry

    lax.fori_loop(0, N_STEPS, body, 0)
    plsc.subcore_barrier()
    pltpu.sync_copy(agg_sh.at[pl.ds(row0, ROWS_PER_TILE)],
                    out_hbm.at[c, pl.ds(row0, ROWS_PER_TILE)])


@functools.cache
def _sc_kernels():
    mesh = plsc.VectorSubcoreMesh(core_axis_name="c", subcore_axis_name="s")
    degrees = functools.partial(
        pl.kernel,
        mesh=mesh,
        out_type=jax.ShapeDtypeStruct((NC, NP, DEGW), jnp.float32),
        scratch_types=[
            pltpu.VMEM((CHUNK,), jnp.int32),
            pltpu.VMEM((CHUNK,), jnp.int32),
            pltpu.VMEM((CHUNK, DEGW), jnp.float32),
            pltpu.VMEM((CHUNK, DEGW), jnp.float32),
            pltpu.VMEM_SHARED((NP, DEGW), jnp.float32),
        ],
    )(_degrees_body)
    segment_sum = functools.partial(
        pl.kernel,
        mesh=mesh,
        out_type=jax.ShapeDtypeStruct((NC, NP, D), jnp.float32),
        scratch_types=[
            pltpu.VMEM((N_CHUNKS, CHUNK), jnp.int32),
            pltpu.VMEM((N_CHUNKS, CHUNK), jnp.int32),
            pltpu.VMEM((CHUNK, D), jnp.float32),
            pltpu.VMEM_SHARED((NP, D), jnp.float32),
            pltpu.SemaphoreType.DMA,
        ],
    )(_segment_sum_body)
    return degrees, segment_sum


_R = 2000


def _mm1_body(x_ref, w_ref, degp_ref, o_ref):
    d = degp_ref[0] + degp_ref[1]
    so = lax.rsqrt(jnp.maximum(d[:, 0:1], 1.0))
    o_ref[...] = jnp.dot(x_ref[...] * so, w_ref[...],
                         preferred_element_type=jnp.float32)


def _mm2_body(aggp_ref, degp_ref, b_ref, w_ref, o_ref):
    agg = aggp_ref[0] + aggp_ref[1]
    d = degp_ref[0] + degp_ref[1]
    si = lax.rsqrt(jnp.maximum(d[:, 1:2], 1.0))
    so = lax.rsqrt(jnp.maximum(d[:, 0:1], 1.0))
    h = jnp.maximum(agg * si + b_ref[...], 0.0)
    o_ref[...] = jnp.dot(h * so, w_ref[...],
                         preferred_element_type=jnp.float32)


def _final_body(aggp_ref, degp_ref, b_ref, wc_ref, bc_ref,
                logits_ref, hg_ref, acc_ref):
    i = pl.program_id(0)
    agg = aggp_ref[0] + aggp_ref[1]
    d = degp_ref[0] + degp_ref[1]
    si = lax.rsqrt(jnp.maximum(d[:, 1:2], 1.0))
    h = jnp.maximum(agg * si + b_ref[...], 0.0)
    part = jnp.sum(h, axis=0, keepdims=True)

    @pl.when(i == 0)
    def _():
        acc_ref[...] = part

    @pl.when(i != 0)
    def _():
        acc_ref[...] = acc_ref[...] + part

    @pl.when(i == (N // _R) - 1)
    def _():
        hg = acc_ref[...] * (1.0 / N)
        hg_ref[...] = hg
        logits_ref[...] = jnp.dot(hg, wc_ref[...],
                                  preferred_element_type=jnp.float32) + bc_ref[...]


def kernel(features, edge_index, W1, b1, W2, b2, Wc, bc):
    _degrees, _segment_sum = _sc_kernels()
    src = edge_index[0].reshape(NW, N_CHUNKS, CHUNK)
    dst = edge_index[1].reshape(NW, N_CHUNKS, CHUNK)
    zD = jnp.zeros((NP, D), jnp.float32)
    z8 = jnp.zeros((NP, DEGW), jnp.float32)
    e_out = jnp.broadcast_to(
        jnp.array([1.0] + [0.0] * (DEGW - 1), jnp.float32), (CHUNK, DEGW))
    e_in = jnp.broadcast_to(
        jnp.array([0.0, 1.0] + [0.0] * (DEGW - 2), jnp.float32), (CHUNK, DEGW))

    degp = _degrees(src, dst, z8, e_out, e_in)

    grid = (N // _R,)
    mm1 = pl.pallas_call(
        _mm1_body,
        grid=grid,
        in_specs=[
            pl.BlockSpec((_R, D), lambda i: (i, 0)),
            pl.BlockSpec((D, D), lambda i: (0, 0)),
            pl.BlockSpec((NC, _R, DEGW), lambda i: (0, i, 0)),
        ],
        out_specs=pl.BlockSpec((_R, D), lambda i: (i, 0)),
        out_shape=jax.ShapeDtypeStruct((N, D), jnp.float32),
    )
    h1pre = mm1(features, W1, degp)

    aggp1 = _segment_sum(h1pre, src, dst, zD)

    mm2 = pl.pallas_call(
        _mm2_body,
        grid=grid,
        in_specs=[
            pl.BlockSpec((NC, _R, D), lambda i: (0, i, 0)),
            pl.BlockSpec((NC, _R, DEGW), lambda i: (0, i, 0)),
            pl.BlockSpec((1, D), lambda i: (0, 0)),
            pl.BlockSpec((D, D), lambda i: (0, 0)),
        ],
        out_specs=pl.BlockSpec((_R, D), lambda i: (i, 0)),
        out_shape=jax.ShapeDtypeStruct((N, D), jnp.float32),
    )
    h2pre = mm2(aggp1, degp, b1.reshape(1, D), W2)

    aggp2 = _segment_sum(h2pre, src, dst, zD)

    fin = pl.pallas_call(
        _final_body,
        grid=grid,
        in_specs=[
            pl.BlockSpec((NC, _R, D), lambda i: (0, i, 0)),
            pl.BlockSpec((NC, _R, DEGW), lambda i: (0, i, 0)),
            pl.BlockSpec((1, D), lambda i: (0, 0)),
            pl.BlockSpec((D, NCLS), lambda i: (0, 0)),
            pl.BlockSpec((1, NCLS), lambda i: (0, 0)),
        ],
        out_specs=[
            pl.BlockSpec((1, NCLS), lambda i: (0, 0)),
            pl.BlockSpec((1, D), lambda i: (0, 0)),
        ],
        out_shape=[
            jax.ShapeDtypeStruct((1, NCLS), jnp.float32),
            jax.ShapeDtypeStruct((1, D), jnp.float32),
        ],
        scratch_shapes=[pltpu.VMEM((1, D), jnp.float32)],
    )
    logits, hg = fin(aggp2, degp, b2.reshape(1, D), Wc, bc.reshape(1, NCLS))
    return (logits, hg)


# speedup vs baseline: 6.9388x; 1.2002x over previous
import functools

import jax
import jax.numpy as jnp
from jax import lax
from jax.experimental import pallas as pl
from jax.experimental.pallas import tpu as pltpu
from jax.experimental.pallas import tpu_sc as plsc

N = 10000
E = 320000
D = 128
NCLS = 16

NC = 2
NS = 16
NW = NC * NS
E_PER_W = E // NW
PAD_PER_W = 240
EP_PER_W = E_PER_W + PAD_PER_W
CHUNK = 128
N_CHUNKS = EP_PER_W // CHUNK
NP = 10240
ROWS_PER_TILE = NP // NS
DEGW = 8


NBLK = 2
CPB = N_CHUNKS // NBLK


def _degrees_body(src_hbm, dst_hbm, z8_hbm, eo_hbm, ei_hbm, out_hbm,
                  sidx, didx, eo_v, ei_v, deg_sh):
    c = lax.axis_index("c")
    s = lax.axis_index("s")
    w = c * NS + s
    row0 = s * ROWS_PER_TILE
    pltpu.sync_copy(z8_hbm.at[pl.ds(row0, ROWS_PER_TILE)],
                    deg_sh.at[pl.ds(row0, ROWS_PER_TILE)])
    pltpu.sync_copy(eo_hbm, eo_v)
    pltpu.sync_copy(ei_hbm, ei_v)
    plsc.subcore_barrier()

    def body(j, carry):
        pltpu.sync_copy(src_hbm.at[w, j], sidx)
        pltpu.sync_copy(dst_hbm.at[w, j], didx)
        pltpu.sync_copy(eo_v, deg_sh.at[sidx], add=True)
        pltpu.sync_copy(ei_v, deg_sh.at[didx], add=True)
        return carry

    lax.fori_loop(0, N_CHUNKS, body, 0)
    plsc.subcore_barrier()
    pltpu.sync_copy(deg_sh.at[pl.ds(row0, ROWS_PER_TILE)],
                    out_hbm.at[c, pl.ds(row0, ROWS_PER_TILE)])


def _segment_sum_body(h_hbm, src_hbm, dst_hbm, z_hbm, out_hbm,
                      sidx, didx, rows, agg_sh, sem_i):
    c = lax.axis_index("c")
    s = lax.axis_index("s")
    w = c * NS + s
    row0 = s * ROWS_PER_TILE
    pltpu.sync_copy(z_hbm.at[pl.ds(row0, ROWS_PER_TILE)],
                    agg_sh.at[pl.ds(row0, ROWS_PER_TILE)])
    plsc.subcore_barrier()

    def blk_body(blk, carry):
        pltpu.async_copy(src_hbm.at[w, blk], sidx, sem_i)
        pltpu.async_copy(dst_hbm.at[w, blk], didx, sem_i)
        pltpu.make_async_copy(src_hbm.at[w, blk], sidx, sem_i).wait()
        pltpu.make_async_copy(dst_hbm.at[w, blk], didx, sem_i).wait()

        def chunk_body(j, carry2):
            pltpu.async_copy(h_hbm.at[sidx.at[j]], rows, sem_i).wait()
            pltpu.sync_copy(rows, agg_sh.at[didx.at[j]], add=True)
            return carry2

        lax.fori_loop(0, CPB, chunk_body, 0)
        return carry

    lax.fori_loop(0, NBLK, blk_body, 0)
    plsc.subcore_barrier()
    pltpu.sync_copy(agg_sh.at[pl.ds(row0, ROWS_PER_TILE)],
                    out_hbm.at[c, pl.ds(row0, ROWS_PER_TILE)])


@functools.cache
def _sc_kernels():
    mesh = plsc.VectorSubcoreMesh(core_axis_name="c", subcore_axis_name="s")
    degrees = functools.partial(
        pl.kernel,
        mesh=mesh,
        out_type=jax.ShapeDtypeStruct((NC, NP, DEGW), jnp.float32),
        scratch_types=[
            pltpu.VMEM((CHUNK,), jnp.int32),
            pltpu.VMEM((CHUNK,), jnp.int32),
            pltpu.VMEM((CHUNK, DEGW), jnp.float32),
            pltpu.VMEM((CHUNK, DEGW), jnp.float32),
            pltpu.VMEM_SHARED((NP, DEGW), jnp.float32),
        ],
    )(_degrees_body)
    segment_sum = functools.partial(
        pl.kernel,
        mesh=mesh,
        out_type=jax.ShapeDtypeStruct((NC, NP, D), jnp.float32),
        scratch_types=[
            pltpu.VMEM((CPB, CHUNK), jnp.int32),
            pltpu.VMEM((CPB, CHUNK), jnp.int32),
            pltpu.VMEM((CHUNK, D), jnp.float32),
            pltpu.VMEM_SHARED((NP, D), jnp.float32),
            pltpu.SemaphoreType.DMA,
        ],
    )(_segment_sum_body)
    return degrees, segment_sum


_R = 2000


def _mm1_body(x_ref, w_ref, degp_ref, o_ref):
    d = degp_ref[0] + degp_ref[1]
    so = lax.rsqrt(jnp.maximum(d[:, 0:1], 1.0))
    o_ref[...] = jnp.dot(x_ref[...] * so, w_ref[...],
                         preferred_element_type=jnp.float32)


def _mm2_body(aggp_ref, degp_ref, b_ref, w_ref, o_ref):
    agg = aggp_ref[0] + aggp_ref[1]
    d = degp_ref[0] + degp_ref[1]
    si = lax.rsqrt(jnp.maximum(d[:, 1:2], 1.0))
    so = lax.rsqrt(jnp.maximum(d[:, 0:1], 1.0))
    h = jnp.maximum(agg * si + b_ref[...], 0.0)
    o_ref[...] = jnp.dot(h * so, w_ref[...],
                         preferred_element_type=jnp.float32)


def _final_body(aggp_ref, degp_ref, b_ref, wc_ref, bc_ref,
                logits_ref, hg_ref, acc_ref):
    i = pl.program_id(0)
    agg = aggp_ref[0] + aggp_ref[1]
    d = degp_ref[0] + degp_ref[1]
    si = lax.rsqrt(jnp.maximum(d[:, 1:2], 1.0))
    h = jnp.maximum(agg * si + b_ref[...], 0.0)
    part = jnp.sum(h, axis=0, keepdims=True)

    @pl.when(i == 0)
    def _():
        acc_ref[...] = part

    @pl.when(i != 0)
    def _():
        acc_ref[...] = acc_ref[...] + part

    @pl.when(i == (N // _R) - 1)
    def _():
        hg = acc_ref[...] * (1.0 / N)
        hg_ref[...] = hg
        logits_ref[...] = jnp.dot(hg, wc_ref[...],
                                  preferred_element_type=jnp.float32) + bc_ref[...]


def kernel(features, edge_index, W1, b1, W2, b2, Wc, bc):
    _degrees, _segment_sum = _sc_kernels()
    ar = jnp.arange(NW * PAD_PER_W, dtype=jnp.int32)
    pad_src = (N + ar % PAD_PER_W).reshape(NW, PAD_PER_W)
    pad_dst = (N + (ar * 7 + 3) % PAD_PER_W).reshape(NW, PAD_PER_W)
    srcp = jnp.concatenate([edge_index[0].reshape(NW, E_PER_W), pad_src], 1)
    dstp = jnp.concatenate([edge_index[1].reshape(NW, E_PER_W), pad_dst], 1)
    src = srcp.reshape(NW, N_CHUNKS, CHUNK)
    dst = dstp.reshape(NW, N_CHUNKS, CHUNK)
    src4 = srcp.reshape(NW, NBLK, CPB, CHUNK)
    dst4 = dstp.reshape(NW, NBLK, CPB, CHUNK)
    zD = jnp.zeros((NP, D), jnp.float32)
    z8 = jnp.zeros((NP, DEGW), jnp.float32)
    e_out = jnp.broadcast_to(
        jnp.array([1.0] + [0.0] * (DEGW - 1), jnp.float32), (CHUNK, DEGW))
    e_in = jnp.broadcast_to(
        jnp.array([0.0, 1.0] + [0.0] * (DEGW - 2), jnp.float32), (CHUNK, DEGW))

    degp = _degrees(src, dst, z8, e_out, e_in)

    grid = (N // _R,)
    mm1 = pl.pallas_call(
        _mm1_body,
        grid=grid,
        in_specs=[
            pl.BlockSpec((_R, D), lambda i: (i, 0)),
            pl.BlockSpec((D, D), lambda i: (0, 0)),
            pl.BlockSpec((NC, _R, DEGW), lambda i: (0, i, 0)),
        ],
        out_specs=pl.BlockSpec((_R, D), lambda i: (i, 0)),
        out_shape=jax.ShapeDtypeStruct((NP, D), jnp.float32),
    )
    h1pre = mm1(features, W1, degp)

    aggp1 = _segment_sum(h1pre, src4, dst4, zD)

    mm2 = pl.pallas_call(
        _mm2_body,
        grid=grid,
        in_specs=[
            pl.BlockSpec((NC, _R, D), lambda i: (0, i, 0)),
            pl.BlockSpec((NC, _R, DEGW), lambda i: (0, i, 0)),
            pl.BlockSpec((1, D), lambda i: (0, 0)),
            pl.BlockSpec((D, D), lambda i: (0, 0)),
        ],
        out_specs=pl.BlockSpec((_R, D), lambda i: (i, 0)),
        out_shape=jax.ShapeDtypeStruct((NP, D), jnp.float32),
    )
    h2pre = mm2(aggp1, degp, b1.reshape(1, D), W2)

    aggp2 = _segment_sum(h2pre, src4, dst4, zD)

    fin = pl.pallas_call(
        _final_body,
        grid=grid,
        in_specs=[
            pl.BlockSpec((NC, _R, D), lambda i: (0, i, 0)),
            pl.BlockSpec((NC, _R, DEGW), lambda i: (0, i, 0)),
            pl.BlockSpec((1, D), lambda i: (0, 0)),
            pl.BlockSpec((D, NCLS), lambda i: (0, 0)),
            pl.BlockSpec((1, NCLS), lambda i: (0, 0)),
        ],
        out_specs=[
            pl.BlockSpec((1, NCLS), lambda i: (0, 0)),
            pl.BlockSpec((1, D), lambda i: (0, 0)),
        ],
        out_shape=[
            jax.ShapeDtypeStruct((1, NCLS), jnp.float32),
            jax.ShapeDtypeStruct((1, D), jnp.float32),
        ],
        scratch_shapes=[pltpu.VMEM((1, D), jnp.float32)],
    )
    logits, hg = fin(aggp2, degp, b2.reshape(1, D), Wc, bc.reshape(1, NCLS))
    return (logits, hg)


# speedup vs baseline: 8.1073x; 1.1684x over previous
import functools

import jax
import jax.numpy as jnp
from jax import lax
from jax.experimental import pallas as pl
from jax.experimental.pallas import tpu as pltpu
from jax.experimental.pallas import tpu_sc as plsc

N = 10000
E = 320000
D = 128
NCLS = 16

NC = 2
NS = 16
NW = NC * NS
E_PER_W = E // NW
PAD_PER_W = 240
EP_PER_W = E_PER_W + PAD_PER_W
CHUNK = 128
N_CHUNKS = EP_PER_W // CHUNK
NP = 10240
ROWS_PER_TILE = NP // NS
DEGW = 8


NBLK = 2
CPB = N_CHUNKS // NBLK


def _degrees_body(src_hbm, dst_hbm, z8_hbm, zn_hbm, out_hbm,
                  sidx, hist, mbuf, outbuf, hist_sh, sem_i):
    c = lax.axis_index("c")
    s = lax.axis_index("s")
    w = c * NS + s
    row0 = s * ROWS_PER_TILE
    ones16 = jnp.ones((16,), jnp.float32)
    lanes = jax.lax.iota(jnp.int32, 16)
    pltpu.sync_copy(z8_hbm.at[pl.ds(row0, ROWS_PER_TILE)], outbuf)

    for col, e_hbm in ((0, src_hbm), (1, dst_hbm)):
        pltpu.sync_copy(zn_hbm, hist)
        for blk in range(NBLK):
            pltpu.async_copy(e_hbm.at[w, blk], sidx, sem_i)
            pltpu.make_async_copy(e_hbm.at[w, blk], sidx, sem_i).wait()

            def chunk_body(j, carry):
                for g in range(CHUNK // 16):
                    idxv = sidx[j, pl.ds(g * 16, 16)]
                    plsc.addupdate_scatter(hist, [idxv], ones16)
                return carry

            lax.fori_loop(0, CPB, chunk_body, 0)

        pltpu.sync_copy(hist, hist_sh.at[s])
        plsc.subcore_barrier()
        pltpu.sync_copy(hist_sh.at[pl.ds(0, NS), pl.ds(row0, ROWS_PER_TILE)],
                        mbuf)

        def merge_body(m, carry):
            acc = mbuf[0, pl.ds(m * 16, 16)]
            for t in range(1, NS):
                acc = acc + mbuf[t, pl.ds(m * 16, 16)]
            ridx = m * 16 + lanes
            cidx = jnp.full((16,), col, jnp.int32)
            plsc.store_scatter(outbuf, [ridx, cidx], acc)
            return carry

        lax.fori_loop(0, ROWS_PER_TILE // 16, merge_body, 0)
        plsc.subcore_barrier()

    pltpu.sync_copy(outbuf, out_hbm.at[c, pl.ds(row0, ROWS_PER_TILE)])


def _segment_sum_body(h_hbm, src_hbm, dst_hbm, z_hbm, out_hbm,
                      sidx, didx, rows, agg_sh, sem_i):
    c = lax.axis_index("c")
    s = lax.axis_index("s")
    w = c * NS + s
    row0 = s * ROWS_PER_TILE
    pltpu.sync_copy(z_hbm.at[pl.ds(row0, ROWS_PER_TILE)],
                    agg_sh.at[pl.ds(row0, ROWS_PER_TILE)])
    plsc.subcore_barrier()

    def blk_body(blk, carry):
        pltpu.async_copy(src_hbm.at[w, blk], sidx, sem_i)
        pltpu.async_copy(dst_hbm.at[w, blk], didx, sem_i)
        pltpu.make_async_copy(src_hbm.at[w, blk], sidx, sem_i).wait()
        pltpu.make_async_copy(dst_hbm.at[w, blk], didx, sem_i).wait()

        def chunk_body(j, carry2):
            pltpu.async_copy(h_hbm.at[sidx.at[j]], rows, sem_i).wait()
            pltpu.sync_copy(rows, agg_sh.at[didx.at[j]], add=True)
            return carry2

        lax.fori_loop(0, CPB, chunk_body, 0)
        return carry

    lax.fori_loop(0, NBLK, blk_body, 0)
    plsc.subcore_barrier()
    pltpu.sync_copy(agg_sh.at[pl.ds(row0, ROWS_PER_TILE)],
                    out_hbm.at[c, pl.ds(row0, ROWS_PER_TILE)])


@functools.cache
def _sc_kernels():
    mesh = plsc.VectorSubcoreMesh(core_axis_name="c", subcore_axis_name="s")
    degrees = functools.partial(
        pl.kernel,
        mesh=mesh,
        compiler_params=pltpu.CompilerParams(needs_layout_passes=False),
        out_type=jax.ShapeDtypeStruct((NC, NP, DEGW), jnp.float32),
        scratch_types=[
            pltpu.VMEM((CPB, CHUNK), jnp.int32),
            pltpu.VMEM((NP,), jnp.float32),
            pltpu.VMEM((NS, ROWS_PER_TILE), jnp.float32),
            pltpu.VMEM((ROWS_PER_TILE, DEGW), jnp.float32),
            pltpu.VMEM_SHARED((NS, NP), jnp.float32),
            pltpu.SemaphoreType.DMA,
        ],
    )(_degrees_body)
    segment_sum = functools.partial(
        pl.kernel,
        mesh=mesh,
        out_type=jax.ShapeDtypeStruct((NC, NP, D), jnp.float32),
        scratch_types=[
            pltpu.VMEM((CPB, CHUNK), jnp.int32),
            pltpu.VMEM((CPB, CHUNK), jnp.int32),
            pltpu.VMEM((CHUNK, D), jnp.float32),
            pltpu.VMEM_SHARED((NP, D), jnp.float32),
            pltpu.SemaphoreType.DMA,
        ],
    )(_segment_sum_body)
    return degrees, segment_sum


_R = 2000


def _mm1_body(x_ref, w_ref, degp_ref, o_ref):
    d = degp_ref[0] + degp_ref[1]
    so = lax.rsqrt(jnp.maximum(d[:, 0:1], 1.0))
    o_ref[...] = jnp.dot(x_ref[...] * so, w_ref[...],
                         preferred_element_type=jnp.float32)


def _mm2_body(aggp_ref, degp_ref, b_ref, w_ref, o_ref):
    agg = aggp_ref[0] + aggp_ref[1]
    d = degp_ref[0] + degp_ref[1]
    si = lax.rsqrt(jnp.maximum(d[:, 1:2], 1.0))
    so = lax.rsqrt(jnp.maximum(d[:, 0:1], 1.0))
    h = jnp.maximum(agg * si + b_ref[...], 0.0)
    o_ref[...] = jnp.dot(h * so, w_ref[...],
                         preferred_element_type=jnp.float32)


def _final_body(aggp_ref, degp_ref, b_ref, wc_ref, bc_ref,
                logits_ref, hg_ref, acc_ref):
    i = pl.program_id(0)
    agg = aggp_ref[0] + aggp_ref[1]
    d = degp_ref[0] + degp_ref[1]
    si = lax.rsqrt(jnp.maximum(d[:, 1:2], 1.0))
    h = jnp.maximum(agg * si + b_ref[...], 0.0)
    part = jnp.sum(h, axis=0, keepdims=True)

    @pl.when(i == 0)
    def _():
        acc_ref[...] = part

    @pl.when(i != 0)
    def _():
        acc_ref[...] = acc_ref[...] + part

    @pl.when(i == (N // _R) - 1)
    def _():
        hg = acc_ref[...] * (1.0 / N)
        hg_ref[...] = hg
        logits_ref[...] = jnp.dot(hg, wc_ref[...],
                                  preferred_element_type=jnp.float32) + bc_ref[...]


def kernel(features, edge_index, W1, b1, W2, b2, Wc, bc):
    _degrees, _segment_sum = _sc_kernels()
    ar = jnp.arange(NW * PAD_PER_W, dtype=jnp.int32)
    pad_src = (N + ar % PAD_PER_W).reshape(NW, PAD_PER_W)
    pad_dst = (N + (ar * 7 + 3) % PAD_PER_W).reshape(NW, PAD_PER_W)
    srcp = jnp.concatenate([edge_index[0].reshape(NW, E_PER_W), pad_src], 1)
    dstp = jnp.concatenate([edge_index[1].reshape(NW, E_PER_W), pad_dst], 1)
    src = srcp.reshape(NW, N_CHUNKS, CHUNK)
    dst = dstp.reshape(NW, N_CHUNKS, CHUNK)
    src4 = srcp.reshape(NW, NBLK, CPB, CHUNK)
    dst4 = dstp.reshape(NW, NBLK, CPB, CHUNK)
    zD = jnp.zeros((NP, D), jnp.float32)
    z8 = jnp.zeros((NP, DEGW), jnp.float32)
    zn = jnp.zeros((NP,), jnp.float32)

    degp = _degrees(src4, dst4, z8, zn)

    grid = (N // _R,)
    mm1 = pl.pallas_call(
        _mm1_body,
        grid=grid,
        in_specs=[
            pl.BlockSpec((_R, D), lambda i: (i, 0)),
            pl.BlockSpec((D, D), lambda i: (0, 0)),
            pl.BlockSpec((NC, _R, DEGW), lambda i: (0, i, 0)),
        ],
        out_specs=pl.BlockSpec((_R, D), lambda i: (i, 0)),
        out_shape=jax.ShapeDtypeStruct((NP, D), jnp.float32),
    )
    h1pre = mm1(features, W1, degp)

    aggp1 = _segment_sum(h1pre, src4, dst4, zD)

    mm2 = pl.pallas_call(
        _mm2_body,
        grid=grid,
        in_specs=[
            pl.BlockSpec((NC, _R, D), lambda i: (0, i, 0)),
            pl.BlockSpec((NC, _R, DEGW), lambda i: (0, i, 0)),
            pl.BlockSpec((1, D), lambda i: (0, 0)),
            pl.BlockSpec((D, D), lambda i: (0, 0)),
        ],
        out_specs=pl.BlockSpec((_R, D), lambda i: (i, 0)),
        out_shape=jax.ShapeDtypeStruct((NP, D), jnp.float32),
    )
    h2pre = mm2(aggp1, degp, b1.reshape(1, D), W2)

    aggp2 = _segment_sum(h2pre, src4, dst4, zD)

    fin = pl.pallas_call(
        _final_body,
        grid=grid,
        in_specs=[
            pl.BlockSpec((NC, _R, D), lambda i: (0, i, 0)),
            pl.BlockSpec((NC, _R, DEGW), lambda i: (0, i, 0)),
            pl.BlockSpec((1, D), lambda i: (0, 0)),
            pl.BlockSpec((D, NCLS), lambda i: (0, 0)),
            pl.BlockSpec((1, NCLS), lambda i: (0, 0)),
        ],
        out_specs=[
            pl.BlockSpec((1, NCLS), lambda i: (0, 0)),
            pl.BlockSpec((1, D), lambda i: (0, 0)),
        ],
        out_shape=[
            jax.ShapeDtypeStruct((1, NCLS), jnp.float32),
            jax.ShapeDtypeStruct((1, D), jnp.float32),
        ],
        scratch_shapes=[pltpu.VMEM((1, D), jnp.float32)],
    )
    logits, hg = fin(aggp2, degp, b2.reshape(1, D), Wc, bc.reshape(1, NCLS))
    return (logits, hg)


# speedup vs baseline: 8.1126x; 1.0007x over previous
import functools

import jax
import jax.numpy as jnp
from jax import lax
from jax.experimental import pallas as pl
from jax.experimental.pallas import tpu as pltpu
from jax.experimental.pallas import tpu_sc as plsc

N = 10000
E = 320000
D = 128
NCLS = 16

NC = 2
NS = 16
NW = NC * NS
E_PER_W = E // NW
PAD_PER_W = 240
EP_PER_W = E_PER_W + PAD_PER_W
CHUNK = 128
N_CHUNKS = EP_PER_W // CHUNK
NP = 10240
ROWS_PER_TILE = NP // NS
DEGW = 8


NBLK = 2
CPB = N_CHUNKS // NBLK


def _degrees_body(src_hbm, dst_hbm, z8_hbm, zn_hbm, out_hbm,
                  sidx, hist, mbuf, outbuf, hist_sh, sem_i):
    c = lax.axis_index("c")
    s = lax.axis_index("s")
    w = c * NS + s
    row0 = s * ROWS_PER_TILE
    ones16 = jnp.ones((16,), jnp.float32)
    lanes = jax.lax.iota(jnp.int32, 16)
    pltpu.sync_copy(z8_hbm.at[pl.ds(row0, ROWS_PER_TILE)], outbuf)

    for col, e_hbm in ((0, src_hbm), (1, dst_hbm)):
        pltpu.sync_copy(zn_hbm, hist)
        for blk in range(NBLK):
            pltpu.async_copy(e_hbm.at[w, blk], sidx, sem_i)
            pltpu.make_async_copy(e_hbm.at[w, blk], sidx, sem_i).wait()

            def chunk_body(j, carry):
                for g in range(CHUNK // 16):
                    idxv = sidx[j, pl.ds(g * 16, 16)]
                    plsc.addupdate_scatter(hist, [idxv], ones16)
                return carry

            lax.fori_loop(0, CPB, chunk_body, 0)

        pltpu.sync_copy(hist, hist_sh.at[s])
        plsc.subcore_barrier()
        pltpu.sync_copy(hist_sh.at[pl.ds(0, NS), pl.ds(row0, ROWS_PER_TILE)],
                        mbuf)

        def merge_body(m, carry):
            acc = mbuf[0, pl.ds(m * 16, 16)]
            for t in range(1, NS):
                acc = acc + mbuf[t, pl.ds(m * 16, 16)]
            ridx = m * 16 + lanes
            cidx = jnp.full((16,), col, jnp.int32)
            plsc.store_scatter(outbuf, [ridx, cidx], acc)
            return carry

        lax.fori_loop(0, ROWS_PER_TILE // 16, merge_body, 0)
        plsc.subcore_barrier()

    pltpu.sync_copy(outbuf, out_hbm.at[c, pl.ds(row0, ROWS_PER_TILE)])


def _segment_sum_body(h_hbm, src_hbm, dst_hbm, z_hbm, out_hbm,
                      sidx, didx, rows, agg_sh, sem_i):
    c = lax.axis_index("c")
    s = lax.axis_index("s")
    w = c * NS + s
    row0 = s * ROWS_PER_TILE
    pltpu.sync_copy(z_hbm.at[pl.ds(row0, ROWS_PER_TILE)],
                    agg_sh.at[pl.ds(row0, ROWS_PER_TILE)])
    plsc.subcore_barrier()

    def blk_body(blk, carry):
        pltpu.async_copy(src_hbm.at[w, blk], sidx, sem_i)
        pltpu.async_copy(dst_hbm.at[w, blk], didx, sem_i)
        pltpu.make_async_copy(src_hbm.at[w, blk], sidx, sem_i).wait()
        pltpu.make_async_copy(dst_hbm.at[w, blk], didx, sem_i).wait()

        def chunk_body(j, carry2):
            pltpu.async_copy(h_hbm.at[sidx.at[j]], rows, sem_i).wait()
            pltpu.sync_copy(rows, agg_sh.at[didx.at[j]], add=True)
            return carry2

        lax.fori_loop(0, CPB, chunk_body, 0)
        return carry

    lax.fori_loop(0, NBLK, blk_body, 0)
    plsc.subcore_barrier()
    pltpu.sync_copy(agg_sh.at[pl.ds(row0, ROWS_PER_TILE)],
                    out_hbm.at[c, pl.ds(row0, ROWS_PER_TILE)])


@functools.cache
def _sc_kernels():
    mesh = plsc.VectorSubcoreMesh(core_axis_name="c", subcore_axis_name="s")
    degrees = functools.partial(
        pl.kernel,
        mesh=mesh,
        compiler_params=pltpu.CompilerParams(needs_layout_passes=False),
        out_type=jax.ShapeDtypeStruct((NC, NP, DEGW), jnp.float32),
        scratch_types=[
            pltpu.VMEM((CPB, CHUNK), jnp.int32),
            pltpu.VMEM((NP,), jnp.float32),
            pltpu.VMEM((NS, ROWS_PER_TILE), jnp.float32),
            pltpu.VMEM((ROWS_PER_TILE, DEGW), jnp.float32),
            pltpu.VMEM_SHARED((NS, NP), jnp.float32),
            pltpu.SemaphoreType.DMA,
        ],
    )(_degrees_body)
    segment_sum = functools.partial(
        pl.kernel,
        mesh=mesh,
        out_type=jax.ShapeDtypeStruct((NC, NP, D), jnp.float32),
        scratch_types=[
            pltpu.VMEM((CPB, CHUNK), jnp.int32),
            pltpu.VMEM((CPB, CHUNK), jnp.int32),
            pltpu.VMEM((CHUNK, D), jnp.float32),
            pltpu.VMEM_SHARED((NP, D), jnp.float32),
            pltpu.SemaphoreType.DMA,
        ],
    )(_segment_sum_body)
    return degrees, segment_sum


_R = 2000


def _mm1_body(x_ref, w_ref, degp_ref, o_ref):
    d = degp_ref[0] + degp_ref[1]
    so = lax.rsqrt(jnp.maximum(d[:, 0:1], 1.0))
    o_ref[...] = jnp.dot(x_ref[...] * so, w_ref[...],
                         preferred_element_type=jnp.float32)


def _mm2_body(aggp_ref, degp_ref, b_ref, w_ref, o_ref):
    agg = aggp_ref[0] + aggp_ref[1]
    d = degp_ref[0] + degp_ref[1]
    si = lax.rsqrt(jnp.maximum(d[:, 1:2], 1.0))
    so = lax.rsqrt(jnp.maximum(d[:, 0:1], 1.0))
    h = jnp.maximum(agg * si + b_ref[...], 0.0)
    o_ref[...] = jnp.dot(h * so, w_ref[...],
                         preferred_element_type=jnp.float32)


def _final_body(aggp_ref, degp_ref, b_ref, wc_ref, bc_ref,
                logits_ref, hg_ref, acc_ref):
    i = pl.program_id(0)
    agg = aggp_ref[0] + aggp_ref[1]
    d = degp_ref[0] + degp_ref[1]
    si = lax.rsqrt(jnp.maximum(d[:, 1:2], 1.0))
    h = jnp.maximum(agg * si + b_ref[...], 0.0)
    part = jnp.sum(h, axis=0, keepdims=True)

    @pl.when(i == 0)
    def _():
        acc_ref[...] = part

    @pl.when(i != 0)
    def _():
        acc_ref[...] = acc_ref[...] + part

    @pl.when(i == (N // _R) - 1)
    def _():
        hg = acc_ref[...] * (1.0 / N)
        hg_ref[...] = hg
        logits_ref[...] = jnp.dot(hg, wc_ref[...],
                                  preferred_element_type=jnp.float32) + bc_ref[...]


def kernel(features, edge_index, W1, b1, W2, b2, Wc, bc):
    _degrees, _segment_sum = _sc_kernels()
    ar = jnp.arange(NW * PAD_PER_W, dtype=jnp.int32)
    pad_src = (N + ar % PAD_PER_W).reshape(NW, PAD_PER_W)
    pad_dst = (N + (ar * 7 + 3) % PAD_PER_W).reshape(NW, PAD_PER_W)
    srcp = jnp.concatenate([edge_index[0].reshape(NW, E_PER_W), pad_src], 1)
    dstp = jnp.concatenate([edge_index[1].reshape(NW, E_PER_W), pad_dst], 1)
    src4 = srcp.reshape(NW, NBLK, CPB, CHUNK)
    dst4 = dstp.reshape(NW, NBLK, CPB, CHUNK)
    zD = jnp.zeros((NP, D), jnp.float32)
    z8 = jnp.zeros((NP, DEGW), jnp.float32)
    zn = jnp.zeros((NP,), jnp.float32)

    degp = _degrees(src4, dst4, z8, zn)

    grid = (N // _R,)
    mm1 = pl.pallas_call(
        _mm1_body,
        grid=grid,
        in_specs=[
            pl.BlockSpec((_R, D), lambda i: (i, 0)),
            pl.BlockSpec((D, D), lambda i: (0, 0)),
            pl.BlockSpec((NC, _R, DEGW), lambda i: (0, i, 0)),
        ],
        out_specs=pl.BlockSpec((_R, D), lambda i: (i, 0)),
        out_shape=jax.ShapeDtypeStruct((NP, D), jnp.float32),
    )
    h1pre = mm1(features, W1, degp)

    aggp1 = _segment_sum(h1pre, src4, dst4, zD)

    mm2 = pl.pallas_call(
        _mm2_body,
        grid=grid,
        in_specs=[
            pl.BlockSpec((NC, _R, D), lambda i: (0, i, 0)),
            pl.BlockSpec((NC, _R, DEGW), lambda i: (0, i, 0)),
            pl.BlockSpec((1, D), lambda i: (0, 0)),
            pl.BlockSpec((D, D), lambda i: (0, 0)),
        ],
        out_specs=pl.BlockSpec((_R, D), lambda i: (i, 0)),
        out_shape=jax.ShapeDtypeStruct((NP, D), jnp.float32),
    )
    h2pre = mm2(aggp1, degp, b1.reshape(1, D), W2)

    aggp2 = _segment_sum(h2pre, src4, dst4, zD)

    fin = pl.pallas_call(
        _final_body,
        grid=grid,
        in_specs=[
            pl.BlockSpec((NC, _R, D), lambda i: (0, i, 0)),
            pl.BlockSpec((NC, _R, DEGW), lambda i: (0, i, 0)),
            pl.BlockSpec((1, D), lambda i: (0, 0)),
            pl.BlockSpec((D, NCLS), lambda i: (0, 0)),
            pl.BlockSpec((1, NCLS), lambda i: (0, 0)),
        ],
        out_specs=[
            pl.BlockSpec((1, NCLS), lambda i: (0, 0)),
            pl.BlockSpec((1, D), lambda i: (0, 0)),
        ],
        out_shape=[
            jax.ShapeDtypeStruct((1, NCLS), jnp.float32),
            jax.ShapeDtypeStruct((1, D), jnp.float32),
        ],
        scratch_shapes=[pltpu.VMEM((1, D), jnp.float32)],
    )
    logits, hg = fin(aggp2, degp, b2.reshape(1, D), Wc, bc.reshape(1, NCLS))
    return (logits, hg)


# speedup vs baseline: 8.1155x; 1.0004x over previous
import functools

import jax
import jax.numpy as jnp
from jax import lax
from jax.experimental import pallas as pl
from jax.experimental.pallas import tpu as pltpu
from jax.experimental.pallas import tpu_sc as plsc

N = 10000
E = 320000
D = 128
NCLS = 16

NC = 2
NS = 16
NW = NC * NS
E_PER_W = E // NW
PAD_PER_W = 240
EP_PER_W = E_PER_W + PAD_PER_W
CHUNK = 128
N_CHUNKS = EP_PER_W // CHUNK
NP = 10240
ROWS_PER_TILE = NP // NS
DEGW = 8


NBLK = 2
CPB = N_CHUNKS // NBLK


def _degrees_body(src_hbm, dst_hbm, z8_hbm, zn_hbm, out_hbm,
                  sidx, hist, mbuf, outbuf, hist_sh, sem_i):
    c = lax.axis_index("c")
    s = lax.axis_index("s")
    w = c * NS + s
    row0 = s * ROWS_PER_TILE
    ones16 = jnp.ones((16,), jnp.float32)
    lanes = jax.lax.iota(jnp.int32, 16)
    pltpu.sync_copy(z8_hbm.at[pl.ds(row0, ROWS_PER_TILE)], outbuf)

    for col, e_hbm in ((0, src_hbm), (1, dst_hbm)):
        pltpu.sync_copy(zn_hbm, hist)
        for blk in range(NBLK):
            pltpu.async_copy(e_hbm.at[w, blk], sidx, sem_i)
            pltpu.make_async_copy(e_hbm.at[w, blk], sidx, sem_i).wait()

            def chunk_body(j, carry):
                for g in range(CHUNK // 16):
                    idxv = sidx[j, pl.ds(g * 16, 16)]
                    plsc.addupdate_scatter(hist, [idxv], ones16)
                return carry

            lax.fori_loop(0, CPB, chunk_body, 0)

        pltpu.sync_copy(hist, hist_sh.at[s])
        plsc.subcore_barrier()
        pltpu.sync_copy(hist_sh.at[pl.ds(0, NS), pl.ds(row0, ROWS_PER_TILE)],
                        mbuf)

        def merge_body(m, carry):
            acc = mbuf[0, pl.ds(m * 16, 16)]
            for t in range(1, NS):
                acc = acc + mbuf[t, pl.ds(m * 16, 16)]
            ridx = m * 16 + lanes
            cidx = jnp.full((16,), col, jnp.int32)
            plsc.store_scatter(outbuf, [ridx, cidx], acc)
            return carry

        lax.fori_loop(0, ROWS_PER_TILE // 16, merge_body, 0)
        plsc.subcore_barrier()

    pltpu.sync_copy(outbuf, out_hbm.at[c, pl.ds(row0, ROWS_PER_TILE)])


def _segment_sum_body(h_hbm, src_hbm, dst_hbm, z_hbm, out_hbm,
                      sidx, didx, rows, agg_sh, sem_i):
    c = lax.axis_index("c")
    s = lax.axis_index("s")
    w = c * NS + s
    row0 = s * ROWS_PER_TILE
    pltpu.sync_copy(z_hbm.at[pl.ds(row0, ROWS_PER_TILE)],
                    agg_sh.at[pl.ds(row0, ROWS_PER_TILE)])
    plsc.subcore_barrier()

    def blk_body(blk, carry):
        pltpu.async_copy(src_hbm.at[w, blk], sidx, sem_i)
        pltpu.async_copy(dst_hbm.at[w, blk], didx, sem_i)
        pltpu.make_async_copy(src_hbm.at[w, blk], sidx, sem_i).wait()
        pltpu.make_async_copy(dst_hbm.at[w, blk], didx, sem_i).wait()

        def chunk_body(j, carry2):
            pltpu.async_copy(h_hbm.at[sidx.at[j]], rows, sem_i).wait()
            pltpu.sync_copy(rows, agg_sh.at[didx.at[j]], add=True)
            return carry2

        lax.fori_loop(0, CPB, chunk_body, 0)
        return carry

    lax.fori_loop(0, NBLK, blk_body, 0)
    plsc.subcore_barrier()
    pltpu.sync_copy(agg_sh.at[pl.ds(row0, ROWS_PER_TILE)],
                    out_hbm.at[c, pl.ds(row0, ROWS_PER_TILE)])


@functools.cache
def _sc_kernels():
    mesh = plsc.VectorSubcoreMesh(core_axis_name="c", subcore_axis_name="s")
    degrees = functools.partial(
        pl.kernel,
        mesh=mesh,
        compiler_params=pltpu.CompilerParams(needs_layout_passes=False),
        out_type=jax.ShapeDtypeStruct((NC, NP, DEGW), jnp.float32),
        scratch_types=[
            pltpu.VMEM((CPB, CHUNK), jnp.int32),
            pltpu.VMEM((NP,), jnp.float32),
            pltpu.VMEM((NS, ROWS_PER_TILE), jnp.float32),
            pltpu.VMEM((ROWS_PER_TILE, DEGW), jnp.float32),
            pltpu.VMEM_SHARED((NS, NP), jnp.float32),
            pltpu.SemaphoreType.DMA,
        ],
    )(_degrees_body)
    segment_sum = functools.partial(
        pl.kernel,
        mesh=mesh,
        out_type=jax.ShapeDtypeStruct((NC, NP, D), jnp.float32),
        scratch_types=[
            pltpu.VMEM((CPB, CHUNK), jnp.int32),
            pltpu.VMEM((CPB, CHUNK), jnp.int32),
            pltpu.VMEM((CHUNK, D), jnp.float32),
            pltpu.VMEM_SHARED((NP, D), jnp.float32),
            pltpu.SemaphoreType.DMA,
        ],
    )(_segment_sum_body)
    return degrees, segment_sum


_R = 10000


def _mm1_body(x_ref, w_ref, degp_ref, o_ref):
    d = degp_ref[0] + degp_ref[1]
    so = lax.rsqrt(jnp.maximum(d[:, 0:1], 1.0))
    o_ref[...] = jnp.dot(x_ref[...] * so, w_ref[...],
                         preferred_element_type=jnp.float32)


def _mm2_body(aggp_ref, degp_ref, b_ref, w_ref, o_ref):
    agg = aggp_ref[0] + aggp_ref[1]
    d = degp_ref[0] + degp_ref[1]
    si = lax.rsqrt(jnp.maximum(d[:, 1:2], 1.0))
    so = lax.rsqrt(jnp.maximum(d[:, 0:1], 1.0))
    h = jnp.maximum(agg * si + b_ref[...], 0.0)
    o_ref[...] = jnp.dot(h * so, w_ref[...],
                         preferred_element_type=jnp.float32)


def _final_body(aggp_ref, degp_ref, b_ref, wc_ref, bc_ref,
                logits_ref, hg_ref, acc_ref):
    i = pl.program_id(0)
    agg = aggp_ref[0] + aggp_ref[1]
    d = degp_ref[0] + degp_ref[1]
    si = lax.rsqrt(jnp.maximum(d[:, 1:2], 1.0))
    h = jnp.maximum(agg * si + b_ref[...], 0.0)
    part = jnp.sum(h, axis=0, keepdims=True)

    @pl.when(i == 0)
    def _():
        acc_ref[...] = part

    @pl.when(i != 0)
    def _():
        acc_ref[...] = acc_ref[...] + part

    @pl.when(i == (N // _R) - 1)
    def _():
        hg = acc_ref[...] * (1.0 / N)
        hg_ref[...] = hg
        logits_ref[...] = jnp.dot(hg, wc_ref[...],
                                  preferred_element_type=jnp.float32) + bc_ref[...]


def kernel(features, edge_index, W1, b1, W2, b2, Wc, bc):
    _degrees, _segment_sum = _sc_kernels()
    ar = jnp.arange(NW * PAD_PER_W, dtype=jnp.int32)
    pad_src = (N + ar % PAD_PER_W).reshape(NW, PAD_PER_W)
    pad_dst = (N + (ar * 7 + 3) % PAD_PER_W).reshape(NW, PAD_PER_W)
    srcp = jnp.concatenate([edge_index[0].reshape(NW, E_PER_W), pad_src], 1)
    dstp = jnp.concatenate([edge_index[1].reshape(NW, E_PER_W), pad_dst], 1)
    src4 = srcp.reshape(NW, NBLK, CPB, CHUNK)
    dst4 = dstp.reshape(NW, NBLK, CPB, CHUNK)
    zD = jnp.zeros((NP, D), jnp.float32)
    z8 = jnp.zeros((NP, DEGW), jnp.float32)
    zn = jnp.zeros((NP,), jnp.float32)

    degp = _degrees(src4, dst4, z8, zn)

    grid = (N // _R,)
    mm1 = pl.pallas_call(
        _mm1_body,
        grid=grid,
        in_specs=[
            pl.BlockSpec((_R, D), lambda i: (i, 0)),
            pl.BlockSpec((D, D), lambda i: (0, 0)),
            pl.BlockSpec((NC, _R, DEGW), lambda i: (0, i, 0)),
        ],
        out_specs=pl.BlockSpec((_R, D), lambda i: (i, 0)),
        out_shape=jax.ShapeDtypeStruct((NP, D), jnp.float32),
    )
    h1pre = mm1(features, W1, degp)

    aggp1 = _segment_sum(h1pre, src4, dst4, zD)

    mm2 = pl.pallas_call(
        _mm2_body,
        grid=grid,
        in_specs=[
            pl.BlockSpec((NC, _R, D), lambda i: (0, i, 0)),
            pl.BlockSpec((NC, _R, DEGW), lambda i: (0, i, 0)),
            pl.BlockSpec((1, D), lambda i: (0, 0)),
            pl.BlockSpec((D, D), lambda i: (0, 0)),
        ],
        out_specs=pl.BlockSpec((_R, D), lambda i: (i, 0)),
        out_shape=jax.ShapeDtypeStruct((NP, D), jnp.float32),
    )
    h2pre = mm2(aggp1, degp, b1.reshape(1, D), W2)

    aggp2 = _segment_sum(h2pre, src4, dst4, zD)

    fin = pl.pallas_call(
        _final_body,
        grid=grid,
        in_specs=[
            pl.BlockSpec((NC, _R, D), lambda i: (0, i, 0)),
            pl.BlockSpec((NC, _R, DEGW), lambda i: (0, i, 0)),
            pl.BlockSpec((1, D), lambda i: (0, 0)),
            pl.BlockSpec((D, NCLS), lambda i: (0, 0)),
            pl.BlockSpec((1, NCLS), lambda i: (0, 0)),
        ],
        out_specs=[
            pl.BlockSpec((1, NCLS), lambda i: (0, 0)),
            pl.BlockSpec((1, D), lambda i: (0, 0)),
        ],
        out_shape=[
            jax.ShapeDtypeStruct((1, NCLS), jnp.float32),
            jax.ShapeDtypeStruct((1, D), jnp.float32),
        ],
        scratch_shapes=[pltpu.VMEM((1, D), jnp.float32)],
    )
    logits, hg = fin(aggp2, degp, b2.reshape(1, D), Wc, bc.reshape(1, NCLS))
    return (logits, hg)
